# baseline (speedup 1.0000x reference)
import functools

import jax
import jax.numpy as jnp
from jax import lax
from jax.experimental import pallas as pl
from jax.experimental.pallas import tpu as pltpu

_P = 4
_K = 2 * _P


def _body(feat_ref, wsal_ref, bsal_ref, a_ref, wm_ref, bm_ref,
          corr_ref, sal_ref, len_ref, salv, posm, maskm, s_ref, mout_ref,
          *, nb, hw, w, t_steps):
    wsal = wsal_ref[...]
    bsal = bsal_ref[...]
    a = 1.0 / (1.0 + jnp.exp(-a_ref[...]))
    wm = wm_ref[...]
    bm = bm_ref[...]
    pt = t_steps * _P
    lane_iota = lax.broadcasted_iota(jnp.int32, (nb, hw), 1)
    lane4 = lax.broadcasted_iota(jnp.int32, (nb, _P), 1)
    sub_iota = lax.broadcasted_iota(jnp.int32, (hw, pt), 0)
    big = jnp.int32(1 << 30)
    rng = range(nb)

    for i in rng:
        sal_i = lax.dot_general(wsal, feat_ref[i], (((1,), (0,)), ((), ())),
                                preferred_element_type=jnp.float32) + bsal
        sal_ref[i] = sal_i
        salv[pl.ds(i, 1), :] = sal_i

    sal = salv[...]
    idxs = []
    for _ in range(_K):
        m = jnp.max(sal, axis=1, keepdims=True)
        idx = jnp.min(jnp.where(sal == m, lane_iota, big),
                      axis=1, keepdims=True)
        idxs.append(idx)
        sal = jnp.where(lane_iota == idx, -jnp.inf, sal)

    def packp(cols):
        v = jnp.zeros((nb, _P), jnp.int32)
        for p, s in enumerate(cols):
            v = jnp.where(lane4 == p, s, v)
        return v

    r0 = packp([q // w for q in idxs[:_P]])
    c0 = packp([q % w for q in idxs[:_P]])
    r1 = packp([q // w for q in idxs[_P:]])
    c1 = packp([q % w for q in idxs[_P:]])
    dr = jnp.abs(r1 - r0)
    dc = jnp.abs(c1 - c0)
    sr = jnp.where(r1 >= r0, 1, -1).astype(jnp.int32)
    sc = jnp.where(c1 >= c0, 1, -1).astype(jnp.int32)

    r, c, err = r0, c0, dr - dc
    active = jnp.ones((nb, _P), jnp.bool_)
    for t in range(t_steps):
        posm[:, pl.ds(t * _P, _P)] = r * w + c
        maskm[:, pl.ds(t * _P, _P)] = jnp.where(active, 1.0, 0.0)
        at_end = (r == r1) & (c == c1)
        nxt = active & (~at_end)
        e2 = 2 * err
        cond1 = e2 > -dc
        cond2 = e2 < dr
        err = jnp.where(nxt, err - jnp.where(cond1, dc, 0)
                        + jnp.where(cond2, dr, 0), err)
        r = jnp.where(nxt, r + jnp.where(cond1, sr, 0), r)
        c = jnp.where(nxt, c + jnp.where(cond2, sc, 0), c)
        active = nxt

    len_ref[...] = jnp.zeros((nb, 128), jnp.float32) + jnp.sum(
        maskm[...], axis=1, keepdims=True)
    mks = []
    for i in rng:
        mk = maskm[pl.ds(i, 1), :]
        mks.append(mk)
        s_ref[i] = jnp.where(sub_iota == posm[pl.ds(i, 1), :], 1.0, 0.0) * mk

    for i in rng:
        gathered = lax.dot_general(s_ref[i], feat_ref[i],
                                   (((0,), (1,)), ((), ())),
                                   preferred_element_type=jnp.float32)
        mout_ref[i] = lax.dot_general(gathered, wm, (((1,), (0,)), ((), ())),
                                      preferred_element_type=jnp.float32) + bm

    hs = [jnp.zeros((_P, wm.shape[0]), jnp.float32) for _ in rng]
    for t in range(t_steps):
        for i in rng:
            hs[i] = a * hs[i] + mout_ref[i, pl.ds(t * _P, _P), :]
            mout_ref[i, pl.ds(t * _P, _P), :] = hs[i]

    for i in rng:
        corr = lax.dot_general(mout_ref[i], s_ref[i], (((0,), (1,)), ((), ())),
                               preferred_element_type=jnp.float32)
        counts = lax.dot_general(mks[i], s_ref[i], (((1,), (1,)), ((), ())),
                                 preferred_element_type=jnp.float32)
        corr_ref[i] = corr * (1.0 / jnp.maximum(counts, 1.0))


@jax.jit
def kernel(features, W_sal, b_sal, A, Wm, bm):
    B, C, H, W = features.shape
    HW = H * W
    T = max(H, W)
    PT = _P * T

    feat = features.reshape(B, C, HW)

    corr, sal, lens = pl.pallas_call(
        functools.partial(_body, nb=B, hw=HW, w=W, t_steps=T),
        grid=(1,),
        in_specs=[
            pl.BlockSpec((B, C, HW), lambda g: (0, 0, 0)),
            pl.BlockSpec((1, C), lambda g: (0, 0)),
            pl.BlockSpec((1, 1), lambda g: (0, 0)),
            pl.BlockSpec((1, C), lambda g: (0, 0)),
            pl.BlockSpec((C, C), lambda g: (0, 0)),
            pl.BlockSpec((1, C), lambda g: (0, 0)),
        ],
        out_specs=[
            pl.BlockSpec((B, C, HW), lambda g: (0, 0, 0)),
            pl.BlockSpec((B, 1, HW), lambda g: (0, 0, 0)),
            pl.BlockSpec((B, 128), lambda g: (0, 0)),
        ],
        out_shape=[
            jax.ShapeDtypeStruct((B, C, HW), jnp.float32),
            jax.ShapeDtypeStruct((B, 1, HW), jnp.float32),
            jax.ShapeDtypeStruct((B, 128), jnp.float32),
        ],
        scratch_shapes=[
            pltpu.VMEM((B, HW), jnp.float32),
            pltpu.VMEM((B, PT), jnp.int32),
            pltpu.VMEM((B, PT), jnp.float32),
            pltpu.VMEM((B, HW, PT), jnp.float32),
            pltpu.VMEM((B, PT, C), jnp.float32),
        ],
    )(feat, W_sal.reshape(1, C), b_sal.reshape(1, 1), A.reshape(1, C),
      Wm, bm.reshape(1, C))

    corrections = corr.reshape(B, C, H, W)
    sal_maps = sal.reshape(B, H, W)
    avg_path_len = jnp.sum(lens[:, 0]) / B
    return (corrections, avg_path_len, sal_maps)


# speedup vs baseline: 1.0628x; 1.0628x over previous
import functools

import jax
import jax.numpy as jnp
from jax import lax
from jax.experimental import pallas as pl
from jax.experimental.pallas import tpu as pltpu

_P = 4
_K = 2 * _P


def _body(feat_ref, wsal_ref, bsal_ref, a_ref, wm_ref, bm_ref,
          corr_ref, sal_ref, len_ref, salv, posm, maskm, s_ref, mout_ref,
          *, nb, hw, w, t_steps):
    wsal = wsal_ref[...]
    bsal = bsal_ref[...]
    a = 1.0 / (1.0 + jnp.exp(-a_ref[...]))
    wm = wm_ref[...]
    bm = bm_ref[...]
    pt = t_steps * _P
    lane_iota = lax.broadcasted_iota(jnp.int32, (nb, hw), 1)
    lane4 = lax.broadcasted_iota(jnp.int32, (nb, _P), 1)
    sub_iota = lax.broadcasted_iota(jnp.int32, (hw, pt), 0)
    big = jnp.int32(1 << 30)
    rng = range(nb)

    for i in rng:
        sal_i = lax.dot_general(wsal, feat_ref[i], (((1,), (0,)), ((), ())),
                                preferred_element_type=jnp.float32) + bsal
        sal_ref[i] = sal_i
        salv[pl.ds(i, 1), :] = sal_i

    sal = salv[...]
    idxs = []
    for _ in range(_K):
        m = jnp.max(sal, axis=1, keepdims=True)
        idx = jnp.min(jnp.where(sal == m, lane_iota, big),
                      axis=1, keepdims=True)
        idxs.append(idx)
        sal = jnp.where(lane_iota == idx, -jnp.inf, sal)

    def packp(cols):
        v = jnp.zeros((nb, _P), jnp.int32)
        for p, s in enumerate(cols):
            v = jnp.where(lane4 == p, s, v)
        return v

    r0 = packp([q // w for q in idxs[:_P]])
    c0 = packp([q % w for q in idxs[:_P]])
    r1 = packp([q // w for q in idxs[_P:]])
    c1 = packp([q % w for q in idxs[_P:]])
    dr = jnp.abs(r1 - r0)
    dc = jnp.abs(c1 - c0)
    sr = jnp.where(r1 >= r0, 1, -1).astype(jnp.int32)
    sc = jnp.where(c1 >= c0, 1, -1).astype(jnp.int32)

    r, c, err = r0, c0, dr - dc
    active = jnp.ones((nb, _P), jnp.bool_)
    for t in range(t_steps):
        posm[:, pl.ds(t * _P, _P)] = r * w + c
        maskm[:, pl.ds(t * _P, _P)] = jnp.where(active, 1.0, 0.0)
        at_end = (r == r1) & (c == c1)
        nxt = active & (~at_end)
        e2 = 2 * err
        cond1 = e2 > -dc
        cond2 = e2 < dr
        err = jnp.where(nxt, err - jnp.where(cond1, dc, 0)
                        + jnp.where(cond2, dr, 0), err)
        r = jnp.where(nxt, r + jnp.where(cond1, sr, 0), r)
        c = jnp.where(nxt, c + jnp.where(cond2, sc, 0), c)
        active = nxt

    len_ref[...] = jnp.zeros((1, 128), jnp.float32) + jnp.sum(maskm[...]) / nb
    mks = []
    for i in rng:
        mk = maskm[pl.ds(i, 1), :]
        mks.append(mk)
        s_ref[i] = jnp.where(sub_iota == posm[pl.ds(i, 1), :], 1.0, 0.0) * mk

    for i in rng:
        gathered = lax.dot_general(s_ref[i], feat_ref[i],
                                   (((0,), (1,)), ((), ())),
                                   preferred_element_type=jnp.float32)
        mout_ref[i] = lax.dot_general(gathered, wm, (((1,), (0,)), ((), ())),
                                      preferred_element_type=jnp.float32) + bm

    hs = [jnp.zeros((_P, wm.shape[0]), jnp.float32) for _ in rng]
    for t in range(t_steps):
        for i in rng:
            hs[i] = a * hs[i] + mout_ref[i, pl.ds(t * _P, _P), :]
            mout_ref[i, pl.ds(t * _P, _P), :] = hs[i]

    for i in rng:
        corr = lax.dot_general(mout_ref[i], s_ref[i], (((0,), (1,)), ((), ())),
                               preferred_element_type=jnp.float32)
        counts = lax.dot_general(mks[i], s_ref[i], (((1,), (1,)), ((), ())),
                                 preferred_element_type=jnp.float32)
        corr_ref[i] = corr * (1.0 / jnp.maximum(counts, 1.0))


@jax.jit
def kernel(features, W_sal, b_sal, A, Wm, bm):
    B, C, H, W = features.shape
    HW = H * W
    T = max(H, W)
    PT = _P * T

    feat = features.reshape(B, C, HW)

    corr, sal, lens = pl.pallas_call(
        functools.partial(_body, nb=B, hw=HW, w=W, t_steps=T),
        grid=(1,),
        in_specs=[
            pl.BlockSpec((B, C, HW), lambda g: (0, 0, 0)),
            pl.BlockSpec((1, C), lambda g: (0, 0)),
            pl.BlockSpec((1, 1), lambda g: (0, 0)),
            pl.BlockSpec((1, C), lambda g: (0, 0)),
            pl.BlockSpec((C, C), lambda g: (0, 0)),
            pl.BlockSpec((1, C), lambda g: (0, 0)),
        ],
        out_specs=[
            pl.BlockSpec((B, C, HW), lambda g: (0, 0, 0)),
            pl.BlockSpec((B, 1, HW), lambda g: (0, 0, 0)),
            pl.BlockSpec((1, 128), lambda g: (0, 0)),
        ],
        out_shape=[
            jax.ShapeDtypeStruct((B, C, HW), jnp.float32),
            jax.ShapeDtypeStruct((B, 1, HW), jnp.float32),
            jax.ShapeDtypeStruct((1, 128), jnp.float32),
        ],
        scratch_shapes=[
            pltpu.VMEM((B, HW), jnp.float32),
            pltpu.VMEM((B, PT), jnp.int32),
            pltpu.VMEM((B, PT), jnp.float32),
            pltpu.VMEM((B, HW, PT), jnp.float32),
            pltpu.VMEM((B, PT, C), jnp.float32),
        ],
    )(feat, W_sal.reshape(1, C), b_sal.reshape(1, 1), A.reshape(1, C),
      Wm, bm.reshape(1, C))

    corrections = corr.reshape(B, C, H, W)
    sal_maps = sal.reshape(B, H, W)
    avg_path_len = lens[0, 0]
    return (corrections, avg_path_len, sal_maps)


# speedup vs baseline: 1.0652x; 1.0023x over previous
import functools

import jax
import jax.numpy as jnp
from jax import lax
from jax.experimental import pallas as pl
from jax.experimental.pallas import tpu as pltpu

_P = 4
_K = 2 * _P


def _body(feat_ref, wsal_ref, bsal_ref, a_ref, wm_ref, bm_ref,
          corr_ref, sal_ref, len_ref, posm, maskm, s_ref, mout_ref,
          *, nb, hw, w, t_steps):
    wsal = wsal_ref[...]
    bsal = bsal_ref[...]
    a = 1.0 / (1.0 + jnp.exp(-a_ref[...]))
    wm = wm_ref[...]
    bm = bm_ref[...]
    pt = t_steps * _P
    lane_iota = lax.broadcasted_iota(jnp.int32, (nb, hw), 1)
    lane4 = lax.broadcasted_iota(jnp.int32, (nb, _P), 1)
    sub_iota = lax.broadcasted_iota(jnp.int32, (hw, pt), 0)
    big = jnp.int32(1 << 30)
    rng = range(nb)

    for i in rng:
        sal_i = lax.dot_general(wsal, feat_ref[i], (((1,), (0,)), ((), ())),
                                preferred_element_type=jnp.float32) + bsal
        sal_ref[pl.ds(i, 1), :] = sal_i

    sal = sal_ref[...]
    idxs = []
    for _ in range(_K):
        m = jnp.max(sal, axis=1, keepdims=True)
        idx = jnp.min(jnp.where(sal == m, lane_iota, big),
                      axis=1, keepdims=True)
        idxs.append(idx)
        sal = jnp.where(lane_iota == idx, -jnp.inf, sal)

    def packp(cols):
        v = jnp.zeros((nb, _P), jnp.int32)
        for p, s in enumerate(cols):
            v = jnp.where(lane4 == p, s, v)
        return v

    r0 = packp([q // w for q in idxs[:_P]])
    c0 = packp([q % w for q in idxs[:_P]])
    r1 = packp([q // w for q in idxs[_P:]])
    c1 = packp([q % w for q in idxs[_P:]])
    dr = jnp.abs(r1 - r0)
    dc = jnp.abs(c1 - c0)
    sr = jnp.where(r1 >= r0, 1, -1).astype(jnp.int32)
    sc = jnp.where(c1 >= c0, 1, -1).astype(jnp.int32)

    r, c, err = r0, c0, dr - dc
    active = jnp.ones((nb, _P), jnp.bool_)
    for t in range(t_steps):
        posm[:, pl.ds(t * _P, _P)] = r * w + c
        maskm[:, pl.ds(t * _P, _P)] = jnp.where(active, 1.0, 0.0)
        at_end = (r == r1) & (c == c1)
        nxt = active & (~at_end)
        e2 = 2 * err
        cond1 = e2 > -dc
        cond2 = e2 < dr
        err = jnp.where(nxt, err - jnp.where(cond1, dc, 0)
                        + jnp.where(cond2, dr, 0), err)
        r = jnp.where(nxt, r + jnp.where(cond1, sr, 0), r)
        c = jnp.where(nxt, c + jnp.where(cond2, sc, 0), c)
        active = nxt

    len_ref[...] = jnp.zeros((1, 128), jnp.float32) + jnp.sum(maskm[...]) / nb
    mks = []
    for i in rng:
        mk = maskm[pl.ds(i, 1), :]
        mks.append(mk)
        s_ref[i] = jnp.where(sub_iota == posm[pl.ds(i, 1), :], 1.0, 0.0) * mk

    for i in rng:
        gathered = lax.dot_general(s_ref[i], feat_ref[i],
                                   (((0,), (1,)), ((), ())),
                                   preferred_element_type=jnp.float32)
        mout_ref[i] = lax.dot_general(gathered, wm, (((1,), (0,)), ((), ())),
                                      preferred_element_type=jnp.float32) + bm

    hs = [jnp.zeros((_P, wm.shape[0]), jnp.float32) for _ in rng]
    for t in range(t_steps):
        for i in rng:
            hs[i] = a * hs[i] + mout_ref[i, pl.ds(t * _P, _P), :]
            mout_ref[i, pl.ds(t * _P, _P), :] = hs[i]

    for i in rng:
        corr = lax.dot_general(mout_ref[i], s_ref[i], (((0,), (1,)), ((), ())),
                               preferred_element_type=jnp.float32)
        counts = lax.dot_general(mks[i], s_ref[i], (((1,), (1,)), ((), ())),
                                 preferred_element_type=jnp.float32)
        corr_ref[i] = corr * (1.0 / jnp.maximum(counts, 1.0))


@jax.jit
def kernel(features, W_sal, b_sal, A, Wm, bm):
    B, C, H, W = features.shape
    HW = H * W
    T = max(H, W)
    PT = _P * T

    feat = features.reshape(B, C, HW)

    corr, sal, lens = pl.pallas_call(
        functools.partial(_body, nb=B, hw=HW, w=W, t_steps=T),
        grid=(1,),
        in_specs=[
            pl.BlockSpec((B, C, HW), lambda g: (0, 0, 0)),
            pl.BlockSpec((1, C), lambda g: (0, 0)),
            pl.BlockSpec((1, 1), lambda g: (0, 0)),
            pl.BlockSpec((1, C), lambda g: (0, 0)),
            pl.BlockSpec((C, C), lambda g: (0, 0)),
            pl.BlockSpec((1, C), lambda g: (0, 0)),
        ],
        out_specs=[
            pl.BlockSpec((B, C, HW), lambda g: (0, 0, 0)),
            pl.BlockSpec((B, HW), lambda g: (0, 0)),
            pl.BlockSpec((1, 128), lambda g: (0, 0)),
        ],
        out_shape=[
            jax.ShapeDtypeStruct((B, C, HW), jnp.float32),
            jax.ShapeDtypeStruct((B, HW), jnp.float32),
            jax.ShapeDtypeStruct((1, 128), jnp.float32),
        ],
        scratch_shapes=[
            pltpu.VMEM((B, PT), jnp.int32),
            pltpu.VMEM((B, PT), jnp.float32),
            pltpu.VMEM((B, HW, PT), jnp.float32),
            pltpu.VMEM((B, PT, C), jnp.float32),
        ],
    )(feat, W_sal.reshape(1, C), b_sal.reshape(1, 1), A.reshape(1, C),
      Wm, bm.reshape(1, C))

    corrections = corr.reshape(B, C, H, W)
    sal_maps = sal.reshape(B, H, W)
    avg_path_len = lens[0, 0]
    return (corrections, avg_path_len, sal_maps)
